# baseline (speedup 1.0000x reference)
import dataclasses
import functools

import jax
import jax.numpy as jnp
from jax import lax
from jax.experimental import pallas as pl
from jax.experimental.pallas import tpu as pltpu
from jax.experimental.pallas import tpu_sc as plsc

_NC = 2
_NS = 16
_NW = _NC * _NS


def _embed_sc(idx, b0, go_f, bp_f, tr_f, d_be, d_go, d_bp, d_tr):
    B = idx.shape[0]
    b_per_w = B // _NW
    mesh = plsc.VectorSubcoreMesh(core_axis_name="c", subcore_axis_name="s")
    cp = pltpu.CompilerParams()
    if "needs_layout_passes" in pltpu.CompilerParams.__dataclass_fields__:
        cp = dataclasses.replace(cp, needs_layout_passes=False)
    if "use_tc_tiling_on_sc" in pltpu.CompilerParams.__dataclass_fields__:
        cp = dataclasses.replace(cp, use_tc_tiling_on_sc=True)

    w_go = (d_go + 7 + 7) // 8 * 8
    w_bp = (d_bp + 7 + 7) // 8 * 8
    w_tr = (d_tr + 7 + 7) // 8 * 8
    n_go, n_bp, n_tr = go_f.shape[0], bp_f.shape[0], tr_f.shape[0]

    @functools.partial(
        pl.kernel,
        mesh=mesh,
        compiler_params=cp,
        out_type=(
            jax.ShapeDtypeStruct((B * d_be,), b0.dtype),
            jax.ShapeDtypeStruct((B * d_go,), go_f.dtype),
            jax.ShapeDtypeStruct((B * d_bp,), bp_f.dtype),
            jax.ShapeDtypeStruct((B * d_tr,), tr_f.dtype),
        ),
        scratch_types=[
            pltpu.VMEM((b_per_w,), jnp.int32),
            pltpu.VMEM((1, d_be), b0.dtype),
            pltpu.VMEM((b_per_w * d_be,), b0.dtype),
            pltpu.VMEM((b_per_w * w_go,), go_f.dtype),
            pltpu.VMEM((b_per_w * w_bp,), bp_f.dtype),
            pltpu.VMEM((b_per_w * w_tr,), tr_f.dtype),
            pltpu.VMEM((b_per_w * d_go,), go_f.dtype),
            pltpu.VMEM((b_per_w * d_bp,), bp_f.dtype),
            pltpu.VMEM((b_per_w * d_tr,), tr_f.dtype),
            pltpu.SemaphoreType.DMA,
        ],
    )
    def k(b0_hbm, go_hbm, bp_hbm, tr_hbm, idx_hbm,
          obe_hbm, ogo_hbm, obp_hbm, otr_hbm,
          idx_v, bsrc_v, be_v, wgo_v, wbp_v, wtr_v, go_v, bp_v, tr_v, sem):
        wid = lax.axis_index("s") * _NC + lax.axis_index("c")
        base = wid * b_per_w
        pltpu.sync_copy(idx_hbm.at[pl.ds(base, b_per_w)], idx_v)

        pltpu.sync_copy(b0_hbm, bsrc_v)
        zeros16 = lax.iota(jnp.int32, 16) * 0

        @pl.loop(0, b_per_w * d_be, step=16)
        def _(off0):
            off = off0 + lax.iota(jnp.int32, 16)
            t = off - (off // d_be) * d_be
            be_v[pl.ds(off0, 16)] = plsc.load_gather(bsrc_v, [zeros16, t])

        pltpu.sync_copy(be_v, obe_hbm.at[pl.ds(base * d_be, b_per_w * d_be)])

        @pl.loop(0, b_per_w, step=16)
        def _(c0):
            v = idx_v[pl.ds(c0, 16)]
            for kk in range(16):
                j = v[kk]
                i = c0 + kk
                a_go = pl.multiple_of(
                    lax.min(lax.bitwise_and(j * d_go, -8), n_go - w_go), 8)
                a_bp = pl.multiple_of(
                    lax.min(lax.bitwise_and(j * d_bp, -8), n_bp - w_bp), 8)
                a_tr = pl.multiple_of(
                    lax.min(lax.bitwise_and(j * d_tr, -8), n_tr - w_tr), 8)
                pltpu.async_copy(go_hbm.at[pl.ds(a_go, w_go)],
                                 wgo_v.at[pl.ds(i * w_go, w_go)], sem)
                pltpu.async_copy(bp_hbm.at[pl.ds(a_bp, w_bp)],
                                 wbp_v.at[pl.ds(i * w_bp, w_bp)], sem)
                pltpu.async_copy(tr_hbm.at[pl.ds(a_tr, w_tr)],
                                 wtr_v.at[pl.ds(i * w_tr, w_tr)], sem)

        @pl.loop(0, b_per_w)
        def _(i):
            pltpu.make_async_copy(go_hbm.at[pl.ds(0, w_go)],
                                  wgo_v.at[pl.ds(i * w_go, w_go)], sem).wait()
            pltpu.make_async_copy(bp_hbm.at[pl.ds(0, w_bp)],
                                  wbp_v.at[pl.ds(i * w_bp, w_bp)], sem).wait()
            pltpu.make_async_copy(tr_hbm.at[pl.ds(0, w_tr)],
                                  wtr_v.at[pl.ds(i * w_tr, w_tr)], sem).wait()

        def realign(w_v, o_v, d, w, n):
            @pl.loop(0, b_per_w * d, step=16)
            def _(o0):
                off = o0 + lax.iota(jnp.int32, 16)
                i = off // d
                t = off - i * d
                jv = plsc.load_gather(idx_v, [i])
                p = jv * d
                a = lax.min(lax.bitwise_and(p, -8), n - w)
                src = i * w + (p - a) + t
                o_v[pl.ds(o0, 16)] = plsc.load_gather(w_v, [src])

        realign(wgo_v, go_v, d_go, w_go, n_go)
        realign(wbp_v, bp_v, d_bp, w_bp, n_bp)
        realign(wtr_v, tr_v, d_tr, w_tr, n_tr)

        pltpu.sync_copy(go_v, ogo_hbm.at[pl.ds(base * d_go, b_per_w * d_go)])
        pltpu.sync_copy(bp_v, obp_hbm.at[pl.ds(base * d_bp, b_per_w * d_bp)])
        pltpu.sync_copy(tr_v, otr_hbm.at[pl.ds(base * d_tr, b_per_w * d_tr)])

    return k(b0, go_f, bp_f, tr_f, idx)


def kernel(idx, betas, global_orient, body_pose, transl):
    B = idx.shape[0]
    idx = idx.astype(jnp.int32)
    d_be = betas.shape[1]
    d_go, d_bp, d_tr = global_orient.shape[1], body_pose.shape[1], transl.shape[1]
    b0 = lax.slice(betas, (0, 0), (1, d_be))
    obe, ogo, obp, otr = _embed_sc(
        idx, b0,
        global_orient.reshape(-1), body_pose.reshape(-1), transl.reshape(-1),
        d_be, d_go, d_bp, d_tr)
    return (obe.reshape(B, d_be), ogo.reshape(B, d_go),
            obp.reshape(B, d_bp), otr.reshape(B, d_tr))


# speedup vs baseline: 5.5386x; 5.5386x over previous
import dataclasses
import functools

import jax
import jax.numpy as jnp
from jax import lax
from jax.experimental import pallas as pl
from jax.experimental.pallas import tpu as pltpu
from jax.experimental.pallas import tpu_sc as plsc

_NC = 2
_NS = 16
_NW = _NC * _NS


def _embed_sc(idx, beT, goT, bpT, trT):
    B = idx.shape[0]
    d_be, n = beT.shape
    d_go, d_bp, d_tr = goT.shape[0], bpT.shape[0], trT.shape[0]
    rows_total = d_be + d_go + d_bp + d_tr
    slots = (rows_total + _NW - 1) // _NW
    e_go = d_be + d_go
    e_bp = e_go + d_bp
    mesh = plsc.VectorSubcoreMesh(core_axis_name="c", subcore_axis_name="s")
    cp = pltpu.CompilerParams()
    if "needs_layout_passes" in pltpu.CompilerParams.__dataclass_fields__:
        cp = dataclasses.replace(cp, needs_layout_passes=False)
    if "use_tc_tiling_on_sc" in pltpu.CompilerParams.__dataclass_fields__:
        cp = dataclasses.replace(cp, use_tc_tiling_on_sc=True)

    @functools.partial(
        pl.kernel,
        mesh=mesh,
        compiler_params=cp,
        out_type=(
            jax.ShapeDtypeStruct((d_be * B,), beT.dtype),
            jax.ShapeDtypeStruct((d_go * B,), goT.dtype),
            jax.ShapeDtypeStruct((d_bp * B,), bpT.dtype),
            jax.ShapeDtypeStruct((d_tr * B,), trT.dtype),
        ),
        scratch_types=[
            pltpu.VMEM((B,), jnp.int32),
            pltpu.VMEM((1, n), beT.dtype),
            pltpu.VMEM((1, 128), beT.dtype),
            pltpu.VMEM((B,), beT.dtype),
        ],
    )
    def k(beT_h, goT_h, bpT_h, trT_h, idx_h,
          obe_h, ogo_h, obp_h, otr_h,
          idx_v, row_v, bcol_v, orow_v):
        wid = lax.axis_index("s") * _NC + lax.axis_index("c")
        pltpu.sync_copy(idx_h, idx_v)
        zeros16 = lax.iota(jnp.int32, 16) * 0

        def gather_row(tbl_h, out_h, c):
            pltpu.sync_copy(tbl_h.at[pl.ds(c, 1)], row_v)

            @pl.loop(0, B, step=16)
            def _(o0):
                v = idx_v[pl.ds(o0, 16)]
                orow_v[pl.ds(o0, 16)] = plsc.load_gather(row_v, [zeros16, v])

            pltpu.sync_copy(orow_v, out_h.at[pl.ds(c * B, B)])

        def bcast_row(c):
            pltpu.sync_copy(beT_h.at[pl.ds(c, 1), pl.ds(0, 128)], bcol_v)
            zv = lax.min(idx_v[pl.ds(0, 16)], 0)
            w = plsc.load_gather(bcol_v, [zv, zv])

            @pl.loop(0, B, step=16)
            def _(o0):
                orow_v[pl.ds(o0, 16)] = w

            pltpu.sync_copy(orow_v, obe_h.at[pl.ds(c * B, B)])

        for s in range(slots):
            m = wid + _NW * s

            @pl.when(m < d_be)
            def _():
                bcast_row(m)

            @pl.when(jnp.logical_and(m >= d_be, m < e_go))
            def _():
                gather_row(goT_h, ogo_h, m - d_be)

            @pl.when(jnp.logical_and(m >= e_go, m < e_bp))
            def _():
                gather_row(bpT_h, obp_h, m - e_go)

            @pl.when(jnp.logical_and(m >= e_bp, m < rows_total))
            def _():
                gather_row(trT_h, otr_h, m - e_bp)

    return k(beT, goT, bpT, trT, idx)


def kernel(idx, betas, global_orient, body_pose, transl):
    B = idx.shape[0]
    idx = idx.astype(jnp.int32)
    obe_f, ogo_f, obp_f, otr_f = _embed_sc(
        idx, betas.T, global_orient.T, body_pose.T, transl.T)

    def unflatten(f, d):
        return f.reshape(d, B).T

    return (unflatten(obe_f, betas.shape[1]),
            unflatten(ogo_f, global_orient.shape[1]),
            unflatten(obp_f, body_pose.shape[1]),
            unflatten(otr_f, transl.shape[1]))


# speedup vs baseline: 6.1570x; 1.1117x over previous
import dataclasses
import functools

import jax
import jax.numpy as jnp
from jax import lax
from jax.experimental import pallas as pl
from jax.experimental.pallas import tpu as pltpu
from jax.experimental.pallas import tpu_sc as plsc

_NC = 2
_NS = 16
_NW = _NC * _NS


def _embed_sc(idx, beT, goT, bpT, trT):
    B = idx.shape[0]
    d_be, n = beT.shape
    d_go, d_bp, d_tr = goT.shape[0], bpT.shape[0], trT.shape[0]
    rows_total = d_be + d_go + d_bp + d_tr
    slots = (rows_total + _NW - 1) // _NW
    e_go = d_be + d_go
    e_bp = e_go + d_bp
    mesh = plsc.VectorSubcoreMesh(core_axis_name="c", subcore_axis_name="s")
    cp = pltpu.CompilerParams()
    if "needs_layout_passes" in pltpu.CompilerParams.__dataclass_fields__:
        cp = dataclasses.replace(cp, needs_layout_passes=False)
    if "use_tc_tiling_on_sc" in pltpu.CompilerParams.__dataclass_fields__:
        cp = dataclasses.replace(cp, use_tc_tiling_on_sc=True)

    @functools.partial(
        pl.kernel,
        mesh=mesh,
        compiler_params=cp,
        out_type=(
            jax.ShapeDtypeStruct((d_be, B), beT.dtype),
            jax.ShapeDtypeStruct((d_go, B), goT.dtype),
            jax.ShapeDtypeStruct((d_bp, B), bpT.dtype),
            jax.ShapeDtypeStruct((d_tr, B), trT.dtype),
        ),
        scratch_types=[
            pltpu.VMEM((B,), jnp.int32),
            pltpu.VMEM((1, n), beT.dtype),
            pltpu.VMEM((1, 128), beT.dtype),
            pltpu.VMEM((B,), beT.dtype),
        ],
    )
    def k(beT_h, goT_h, bpT_h, trT_h, idx_h,
          obe_h, ogo_h, obp_h, otr_h,
          idx_v, row_v, bcol_v, orow_v):
        wid = lax.axis_index("s") * _NC + lax.axis_index("c")
        pltpu.sync_copy(idx_h, idx_v)
        zeros16 = lax.iota(jnp.int32, 16) * 0

        def gather_row(tbl_h, out_h, c):
            pltpu.sync_copy(tbl_h.at[pl.ds(c, 1)], row_v)

            @pl.loop(0, B, step=16)
            def _(o0):
                v = idx_v[pl.ds(o0, 16)]
                orow_v[pl.ds(o0, 16)] = plsc.load_gather(row_v, [zeros16, v])

            pltpu.sync_copy(orow_v, out_h.at[c])

        def bcast_row(c):
            pltpu.sync_copy(beT_h.at[pl.ds(c, 1), pl.ds(0, 128)], bcol_v)
            zv = lax.min(idx_v[pl.ds(0, 16)], 0)
            w = plsc.load_gather(bcol_v, [zv, zv])

            @pl.loop(0, B, step=16)
            def _(o0):
                orow_v[pl.ds(o0, 16)] = w

            pltpu.sync_copy(orow_v, obe_h.at[c])

        for s in range(slots):
            m = wid + _NW * s

            @pl.when(m < d_be)
            def _():
                bcast_row(m)

            @pl.when(jnp.logical_and(m >= d_be, m < e_go))
            def _():
                gather_row(goT_h, ogo_h, m - d_be)

            @pl.when(jnp.logical_and(m >= e_go, m < e_bp))
            def _():
                gather_row(bpT_h, obp_h, m - e_go)

            @pl.when(jnp.logical_and(m >= e_bp, m < rows_total))
            def _():
                gather_row(trT_h, otr_h, m - e_bp)

    return k(beT, goT, bpT, trT, idx)


def kernel(idx, betas, global_orient, body_pose, transl):
    B = idx.shape[0]
    idx = idx.astype(jnp.int32)
    obeT, ogoT, obpT, otrT = _embed_sc(
        idx, betas.T, global_orient.T, body_pose.T, transl.T)
    return (obeT.T, ogoT.T, obpT.T, otrT.T)
